# baseline (speedup 1.0000x reference)
import jax
import jax.numpy as jnp
from jax.experimental import pallas as pl
from jax.experimental.pallas import tpu as pltpu


def _mlp_kernel(x_ref, w1_ref, b1_ref, w2_ref, b2_ref, w3_ref, b3_ref, o_ref):
    tb2 = x_ref.shape[0]
    x = x_ref[...].astype(jnp.bfloat16)
    h1 = jnp.dot(x, w1_ref[...], preferred_element_type=jnp.float32)
    h1 = jnp.maximum(h1 + b1_ref[...], 0.0).astype(jnp.bfloat16)
    h2 = jnp.dot(h1, w2_ref[...], preferred_element_type=jnp.float32)
    h2 = jnp.maximum(h2 + b2_ref[...], 0.0).astype(jnp.bfloat16)
    q = jnp.dot(h2, w3_ref[...], preferred_element_type=jnp.float32)
    q = q + b3_ref[...]
    o_ref[pl.Slice(0, tb2, 2), :] = q[:, 0:9].astype(o_ref.dtype)
    o_ref[pl.Slice(1, tb2, 2), :] = q[:, 9:18].astype(o_ref.dtype)


def _block_diag2(w):
    k, n = w.shape
    z = jnp.zeros((k, n), w.dtype)
    return jnp.concatenate(
        [jnp.concatenate([w, z], axis=1), jnp.concatenate([z, w], axis=1)],
        axis=0)


def kernel(x, w1t, b1, w2t, b2, w3t, b3):
    B = x.shape[0]
    x2 = x.reshape(B // 2, 18)
    R = x2.shape[0]
    tb2 = min(8192, R)
    n_blk = pl.cdiv(R, tb2)

    w1p = _block_diag2(w1t).astype(jnp.bfloat16)
    w2p = _block_diag2(w2t).astype(jnp.bfloat16)
    w3p = _block_diag2(w3t).astype(jnp.bfloat16)
    b1p = jnp.concatenate([b1, b1], axis=1)
    b2p = jnp.concatenate([b2, b2], axis=1)
    b3p = jnp.concatenate([b3, b3], axis=1)

    const = lambda shape: pl.BlockSpec(shape, lambda i: (0, 0))

    flops = 2 * B * (9 * 128 + 128 * 64 + 64 * 9)
    bytes_accessed = 4 * B * 9 * 2 + 2 * (18 * 256 + 256 * 128 + 128 * 18) \
        + 4 * (256 + 128 + 18)

    return pl.pallas_call(
        _mlp_kernel,
        out_shape=jax.ShapeDtypeStruct((B, 9), x.dtype),
        grid=(n_blk,),
        in_specs=[
            pl.BlockSpec((tb2, 18), lambda i: (i, 0)),
            const(w1p.shape), const(b1p.shape),
            const(w2p.shape), const(b2p.shape),
            const(w3p.shape), const(b3p.shape),
        ],
        out_specs=pl.BlockSpec((2 * tb2, 9), lambda i: (i, 0)),
        compiler_params=pltpu.CompilerParams(
            dimension_semantics=("parallel",),
        ),
        cost_estimate=pl.CostEstimate(flops=flops, transcendentals=0,
                                      bytes_accessed=bytes_accessed),
    )(x2, w1p, b1p, w2p, b2p, w3p, b3p)


# speedup vs baseline: 1.1548x; 1.1548x over previous
import jax
import jax.numpy as jnp
from jax.experimental import pallas as pl
from jax.experimental.pallas import tpu as pltpu


def _mlp_kernel(x_ref, w1_ref, b1_ref, w2_ref, b2_ref, w3_ref, b3_ref, o_ref):
    x = x_ref[...].astype(jnp.bfloat16)
    h1 = jnp.dot(x, w1_ref[...], preferred_element_type=jnp.float32)
    h1 = jnp.maximum(h1 + b1_ref[...], 0.0).astype(jnp.bfloat16)
    h2 = jnp.dot(h1, w2_ref[...], preferred_element_type=jnp.float32)
    h2 = jnp.maximum(h2 + b2_ref[...], 0.0).astype(jnp.bfloat16)
    q = jnp.dot(h2, w3_ref[...], preferred_element_type=jnp.float32)
    o_ref[...] = (q + b3_ref[...]).astype(o_ref.dtype)


def kernel(x, w1t, b1, w2t, b2, w3t, b3):
    B = x.shape[0]
    tb = min(16384, B)
    n_blk = pl.cdiv(B, tb)

    w1b = w1t.astype(jnp.bfloat16)
    w2b = w2t.astype(jnp.bfloat16)
    w3b = w3t.astype(jnp.bfloat16)

    const = lambda shape: pl.BlockSpec(shape, lambda i: (0, 0))

    flops = 2 * B * (9 * 128 + 128 * 64 + 64 * 9)
    bytes_accessed = 4 * B * 9 * 2 + 2 * (9 * 128 + 128 * 64 + 64 * 9) \
        + 4 * (128 + 64 + 9)

    return pl.pallas_call(
        _mlp_kernel,
        out_shape=jax.ShapeDtypeStruct((B, 9), x.dtype),
        grid=(n_blk,),
        in_specs=[
            pl.BlockSpec((tb, 9), lambda i: (i, 0)),
            const(w1b.shape), const(b1.shape),
            const(w2b.shape), const(b2.shape),
            const(w3b.shape), const(b3.shape),
        ],
        out_specs=pl.BlockSpec((tb, 9), lambda i: (i, 0)),
        compiler_params=pltpu.CompilerParams(
            dimension_semantics=("parallel",),
        ),
        cost_estimate=pl.CostEstimate(flops=flops, transcendentals=0,
                                      bytes_accessed=bytes_accessed),
    )(x, w1b, b1, w2b, b2, w3b, b3)
